# baseline (speedup 1.0000x reference)
import functools

import jax
import jax.numpy as jnp
from jax import lax
from jax.experimental import pallas as pl
from jax.experimental.pallas import tpu as pltpu
from jax.experimental.pallas import tpu_sc as plsc

N_NODES = 10000
N_EDGES = 320000
D = 128
NC = 2
NS = 16
NW = NC * NS
CHUNK = 64
SB = 32
NSUPER = 5
E_PER_W = CHUNK * SB * NSUPER
E_PAD = E_PER_W * NW
ROWS_PER_TILE = 624


def _lane_bcast(vv, l):
    return lax.gather(
        vv,
        jnp.full((16, 1), l, jnp.int32),
        lax.GatherDimensionNumbers(
            offset_dims=(), collapsed_slice_dims=(0,), start_index_map=(0,)
        ),
        slice_sizes=(1,),
        mode=lax.GatherScatterMode.PROMISE_IN_BOUNDS,
    )


def _sc_partials(all_emb, packed, vals4):
    mesh = plsc.VectorSubcoreMesh(
        core_axis_name="c", subcore_axis_name="s", num_cores=NC, num_subcores=NS
    )

    @functools.partial(
        pl.kernel,
        out_type=jax.ShapeDtypeStruct((NC * N_NODES, D), jnp.float32),
        mesh=mesh,
        scratch_types=[
            pltpu.VMEM((2, SB, CHUNK), jnp.int32),
            pltpu.VMEM((SB, CHUNK), jnp.float32),
            pltpu.VMEM((CHUNK, D), jnp.float32),
            pltpu.VMEM((CHUNK, D), jnp.float32),
            pltpu.VMEM_SHARED((N_NODES, D), jnp.float32),
            pltpu.SemaphoreType.DMA,
            pltpu.SemaphoreType.DMA,
            pltpu.SemaphoreType.DMA,
        ],
    )
    def body(emb_hbm, pk_hbm, vals_hbm, out_hbm,
             stg, vstg, rows0, rows1, acc_sh, ssem, gsem0, gsem1):
        cid = lax.axis_index("c")
        sid = lax.axis_index("s")
        wid = sid * NC + cid

        zeros16 = jnp.zeros((16,), jnp.float32)

        def zero_row(i, carry):
            for j in range(D // 16):
                rows0[i, pl.ds(j * 16, 16)] = zeros16
            return carry

        lax.fori_loop(0, CHUNK, zero_row, 0)
        zb = sid * ROWS_PER_TILE
        for kk in range(ROWS_PER_TILE // CHUNK):
            pltpu.sync_copy(rows0, acc_sh.at[pl.ds(zb + kk * CHUNK, CHUNK)])
        rem = ROWS_PER_TILE % CHUNK
        pltpu.sync_copy(
            rows0.at[pl.ds(0, rem)],
            acc_sh.at[pl.ds(zb + ROWS_PER_TILE - rem, rem)],
        )

        @pl.when(sid == NS - 1)
        def _zero_tail():
            pltpu.sync_copy(
                rows0.at[pl.ds(0, 16)], acc_sh.at[pl.ds(NS * ROWS_PER_TILE, 16)]
            )

        plsc.subcore_barrier()

        bufs = (rows0, rows1)
        sems = (gsem0, gsem1)

        def fire(j, buf, sem):
            pltpu.async_copy(emb_hbm.at[stg.at[0, j]], buf, sem)

        def gwait(j, buf, sem):
            pltpu.make_async_copy(emb_hbm.at[stg.at[0, j]], buf, sem).wait()

        def process(j, buf):
            def group_body(g, c2):
                vv = vstg[j, pl.ds(g * 16, 16)]
                for l in range(16):
                    bv = _lane_bcast(vv, l)
                    r = g * 16 + l
                    for jj in range(D // 16):
                        sl = pl.ds(jj * 16, 16)
                        buf[r, sl] = buf[r, sl] * bv
                return c2

            lax.fori_loop(0, CHUNK // 16, group_body, 0)
            pltpu.sync_copy(buf, acc_sh.at[stg.at[1, j]], add=True)

        def super_body(s, carry):
            d_pk = pltpu.async_copy(pk_hbm.at[wid, s], stg, ssem)
            d_v = pltpu.async_copy(vals_hbm.at[wid, s], vstg, ssem)
            d_pk.wait()
            d_v.wait()
            fire(0, bufs[0], sems[0])

            def inner(o, c2):
                for b in range(2):
                    j = 2 * o + b
                    gwait(j, bufs[b], sems[b])

                    @pl.when(j + 1 < SB)
                    def _fire_next():
                        fire(j + 1, bufs[1 - b], sems[1 - b])

                    process(j, bufs[b])
                return c2

            lax.fori_loop(0, SB // 2, inner, 0)
            return carry

        lax.fori_loop(0, NSUPER, super_body, 0)
        plsc.subcore_barrier()

        off = cid * N_NODES + sid * ROWS_PER_TILE
        pltpu.sync_copy(
            acc_sh.at[pl.ds(sid * ROWS_PER_TILE, ROWS_PER_TILE)],
            out_hbm.at[pl.ds(off, ROWS_PER_TILE)],
        )

        @pl.when(sid == NS - 1)
        def _copy_tail():
            tail = NS * ROWS_PER_TILE
            pltpu.sync_copy(
                acc_sh.at[pl.ds(tail, 16)],
                out_hbm.at[pl.ds(cid * N_NODES + tail, 16)],
            )

    return body(all_emb, packed, vals4)


def _tc_sum(p0, p1):
    def add_body(a_ref, b_ref, o_ref):
        o_ref[...] = a_ref[...] + b_ref[...]

    blk = 1000
    return pl.pallas_call(
        add_body,
        grid=(N_NODES // blk,),
        in_specs=[
            pl.BlockSpec((blk, D), lambda i: (i, 0)),
            pl.BlockSpec((blk, D), lambda i: (i, 0)),
        ],
        out_specs=pl.BlockSpec((blk, D), lambda i: (i, 0)),
        out_shape=jax.ShapeDtypeStruct((N_NODES, D), jnp.float32),
    )(p0, p1)


def kernel(users_emb, items_emb, edge_index, edge_vals):
    num_user = users_emb.shape[0]
    all_emb = jnp.concatenate([users_emb, items_emb], axis=0)
    pad = E_PAD - N_EDGES
    dst = jnp.pad(edge_index[0].astype(jnp.int32), (0, pad))
    src = jnp.pad(edge_index[1].astype(jnp.int32), (0, pad))
    vals4 = jnp.pad(edge_vals, (0, pad)).reshape(NW, NSUPER, SB, CHUNK)
    packed = jnp.stack(
        [src.reshape(NW, NSUPER, SB, CHUNK),
         dst.reshape(NW, NSUPER, SB, CHUNK)],
        axis=2,
    )
    partials = _sc_partials(all_emb, packed, vals4)
    out = _tc_sum(partials[:N_NODES], partials[N_NODES:])
    return (out[:num_user], out[num_user:])


# speedup vs baseline: 1.2437x; 1.2437x over previous
import functools

import jax
import jax.numpy as jnp
from jax import lax
from jax.experimental import pallas as pl
from jax.experimental.pallas import tpu as pltpu
from jax.experimental.pallas import tpu_sc as plsc

N_NODES = 10000
N_EDGES = 320000
D = 128
NC = 2
NS = 16
NW = NC * NS
CHUNK = 128
SB = 20
NSUPER = 4
E_PER_W = CHUNK * SB * NSUPER
E_PAD = E_PER_W * NW
ROWS_PER_TILE = 624


def _lane_bcast(vv, l):
    return lax.gather(
        vv,
        jnp.full((16, 1), l, jnp.int32),
        lax.GatherDimensionNumbers(
            offset_dims=(), collapsed_slice_dims=(0,), start_index_map=(0,)
        ),
        slice_sizes=(1,),
        mode=lax.GatherScatterMode.PROMISE_IN_BOUNDS,
    )


def _sc_partials(all_emb, packed, vals4):
    mesh = plsc.VectorSubcoreMesh(
        core_axis_name="c", subcore_axis_name="s", num_cores=NC, num_subcores=NS
    )

    @functools.partial(
        pl.kernel,
        out_type=jax.ShapeDtypeStruct((NC * N_NODES, D), jnp.float32),
        mesh=mesh,
        scratch_types=[
            pltpu.VMEM((2, SB, CHUNK), jnp.int32),
            pltpu.VMEM((SB, CHUNK), jnp.float32),
            pltpu.VMEM((CHUNK, D), jnp.float32),
            pltpu.VMEM((CHUNK, D), jnp.float32),
            pltpu.VMEM_SHARED((N_NODES, D), jnp.float32),
            pltpu.SemaphoreType.DMA,
            pltpu.SemaphoreType.DMA,
            pltpu.SemaphoreType.DMA,
        ],
    )
    def body(emb_hbm, pk_hbm, vals_hbm, out_hbm,
             stg, vstg, rows0, rows1, acc_sh, ssem, gsem0, gsem1):
        cid = lax.axis_index("c")
        sid = lax.axis_index("s")
        wid = sid * NC + cid

        zeros16 = jnp.zeros((16,), jnp.float32)

        def zero_row(i, carry):
            for j in range(D // 16):
                rows0[i, pl.ds(j * 16, 16)] = zeros16
            return carry

        lax.fori_loop(0, CHUNK, zero_row, 0)
        zb = sid * ROWS_PER_TILE
        for kk in range(ROWS_PER_TILE // CHUNK):
            pltpu.sync_copy(rows0, acc_sh.at[pl.ds(zb + kk * CHUNK, CHUNK)])
        rem = ROWS_PER_TILE % CHUNK
        if rem:
            pltpu.sync_copy(
                rows0.at[pl.ds(0, rem)],
                acc_sh.at[pl.ds(zb + ROWS_PER_TILE - rem, rem)],
            )

        @pl.when(sid == NS - 1)
        def _zero_tail():
            pltpu.sync_copy(
                rows0.at[pl.ds(0, 16)], acc_sh.at[pl.ds(NS * ROWS_PER_TILE, 16)]
            )

        plsc.subcore_barrier()

        bufs = (rows0, rows1)
        sems = (gsem0, gsem1)

        def fire(j, buf, sem):
            pltpu.async_copy(emb_hbm.at[stg.at[0, j]], buf, sem)

        def gwait(j, buf, sem):
            pltpu.make_async_copy(emb_hbm.at[stg.at[0, j]], buf, sem).wait()

        def process(j, buf):
            def group_body(g, c2):
                vv = vstg[j, pl.ds(g * 16, 16)]
                for l in range(16):
                    bv = _lane_bcast(vv, l)
                    r = g * 16 + l
                    for jj in range(D // 16):
                        sl = pl.ds(jj * 16, 16)
                        buf[r, sl] = buf[r, sl] * bv
                return c2

            lax.fori_loop(0, CHUNK // 16, group_body, 0)
            pltpu.sync_copy(buf, acc_sh.at[stg.at[1, j]], add=True)

        def super_body(s, carry):
            d_pk = pltpu.async_copy(pk_hbm.at[wid, s], stg, ssem)
            d_v = pltpu.async_copy(vals_hbm.at[wid, s], vstg, ssem)
            d_pk.wait()
            d_v.wait()
            fire(0, bufs[0], sems[0])

            def inner(o, c2):
                for b in range(2):
                    j = 2 * o + b
                    gwait(j, bufs[b], sems[b])

                    @pl.when(j + 1 < SB)
                    def _fire_next():
                        fire(j + 1, bufs[1 - b], sems[1 - b])

                    process(j, bufs[b])
                return c2

            lax.fori_loop(0, SB // 2, inner, 0)
            return carry

        lax.fori_loop(0, NSUPER, super_body, 0)
        plsc.subcore_barrier()

        off = cid * N_NODES + sid * ROWS_PER_TILE
        pltpu.sync_copy(
            acc_sh.at[pl.ds(sid * ROWS_PER_TILE, ROWS_PER_TILE)],
            out_hbm.at[pl.ds(off, ROWS_PER_TILE)],
        )

        @pl.when(sid == NS - 1)
        def _copy_tail():
            tail = NS * ROWS_PER_TILE
            pltpu.sync_copy(
                acc_sh.at[pl.ds(tail, 16)],
                out_hbm.at[pl.ds(cid * N_NODES + tail, 16)],
            )

    return body(all_emb, packed, vals4)


def _tc_sum(p0, p1):
    def add_body(a_ref, b_ref, o_ref):
        o_ref[...] = a_ref[...] + b_ref[...]

    blk = 1000
    return pl.pallas_call(
        add_body,
        grid=(N_NODES // blk,),
        in_specs=[
            pl.BlockSpec((blk, D), lambda i: (i, 0)),
            pl.BlockSpec((blk, D), lambda i: (i, 0)),
        ],
        out_specs=pl.BlockSpec((blk, D), lambda i: (i, 0)),
        out_shape=jax.ShapeDtypeStruct((N_NODES, D), jnp.float32),
    )(p0, p1)


def kernel(users_emb, items_emb, edge_index, edge_vals):
    num_user = users_emb.shape[0]
    all_emb = jnp.concatenate([users_emb, items_emb], axis=0)
    pad = E_PAD - N_EDGES
    dst = jnp.pad(edge_index[0].astype(jnp.int32), (0, pad))
    src = jnp.pad(edge_index[1].astype(jnp.int32), (0, pad))
    vals4 = jnp.pad(edge_vals, (0, pad)).reshape(NW, NSUPER, SB, CHUNK)
    packed = jnp.stack(
        [src.reshape(NW, NSUPER, SB, CHUNK),
         dst.reshape(NW, NSUPER, SB, CHUNK)],
        axis=2,
    )
    partials = _sc_partials(all_emb, packed, vals4)
    out = _tc_sum(partials[:N_NODES], partials[N_NODES:])
    return (out[:num_user], out[num_user:])
